# baseline (speedup 1.0000x reference)
import jax
import jax.numpy as jnp
from jax.experimental import pallas as pl


def _add_kernel(x_ref, pos_ref, o_ref):
    o_ref[...] = x_ref[...] + pos_ref[...]


def kernel(x, positions):
    B, S, D = x.shape
    pos = positions[0, :S, :]
    x2 = x.reshape(B * S, D)

    BS = 512
    n_pos_blocks = S // BS
    grid = ((B * S) // BS,)

    out = pl.pallas_call(
        _add_kernel,
        grid=grid,
        in_specs=[
            pl.BlockSpec((BS, D), lambda i: (i, 0)),
            pl.BlockSpec((BS, D), lambda i, n=n_pos_blocks: (i % n, 0)),
        ],
        out_specs=pl.BlockSpec((BS, D), lambda i: (i, 0)),
        out_shape=jax.ShapeDtypeStruct((B * S, D), x.dtype),
    )(x2, pos)
    return out.reshape(B, S, D)


# speedup vs baseline: 1.3161x; 1.3161x over previous
import jax
import jax.numpy as jnp
from jax.experimental import pallas as pl


def _add_kernel(x_ref, pos_ref, o_ref):
    o_ref[...] = x_ref[...] + pos_ref[...]


def kernel(x, positions):
    B, S, D = x.shape
    pos = positions[:, :S, :]

    BS = 256
    grid = (S // BS,)

    out = pl.pallas_call(
        _add_kernel,
        grid=grid,
        in_specs=[
            pl.BlockSpec((B, BS, D), lambda i: (0, i, 0)),
            pl.BlockSpec((1, BS, D), lambda i: (0, i, 0)),
        ],
        out_specs=pl.BlockSpec((B, BS, D), lambda i: (0, i, 0)),
        out_shape=jax.ShapeDtypeStruct((B, S, D), x.dtype),
    )(x, pos)
    return out


# speedup vs baseline: 1.3388x; 1.0173x over previous
import jax
import jax.numpy as jnp
from jax.experimental import pallas as pl


def _add_kernel(x_ref, pos_ref, o_ref):
    o_ref[...] = x_ref[...] + pos_ref[...]


def kernel(x, positions):
    B, S, D = x.shape
    pos = positions[:, :S, :]

    BS = 512
    grid = (S // BS,)

    out = pl.pallas_call(
        _add_kernel,
        grid=grid,
        in_specs=[
            pl.BlockSpec((B, BS, D), lambda i: (0, i, 0)),
            pl.BlockSpec((1, BS, D), lambda i: (0, i, 0)),
        ],
        out_specs=pl.BlockSpec((B, BS, D), lambda i: (0, i, 0)),
        out_shape=jax.ShapeDtypeStruct((B, S, D), x.dtype),
    )(x, pos)
    return out


# speedup vs baseline: 1.3732x; 1.0257x over previous
import jax
import jax.numpy as jnp
from jax.experimental import pallas as pl


def _make_kernel(BS, S):
    n_pos_blocks = S // BS

    def _add_kernel(x_ref, pos_ref, o_ref):
        i = pl.program_id(0)
        base = (i % n_pos_blocks) * BS
        o_ref[...] = x_ref[...] + pos_ref[pl.ds(base, BS), :]

    return _add_kernel


def kernel(x, positions):
    B, S, D = x.shape
    pos = positions[0, :S, :]
    x2 = x.reshape(B * S, D)

    BS = 2048
    grid = ((B * S) // BS,)

    out = pl.pallas_call(
        _make_kernel(BS, S),
        grid=grid,
        in_specs=[
            pl.BlockSpec((BS, D), lambda i: (i, 0)),
            pl.BlockSpec((S, D), lambda i: (0, 0)),
        ],
        out_specs=pl.BlockSpec((BS, D), lambda i: (i, 0)),
        out_shape=jax.ShapeDtypeStruct((B * S, D), x.dtype),
    )(x2, pos)
    return out.reshape(B, S, D)
